# baseline (device time: 17794 ns/iter reference)
import math

import jax
import jax.numpy as jnp
from jax import lax
from jax.experimental import pallas as pl
from jax.experimental.pallas import tpu as pltpu

N_DEV = 4


def kernel(q, k, v):
    s_per, d = q.shape
    half = s_per // 2
    sc_rows = s_per // d

    def body(
        q_ref,
        k_ref,
        v_ref,
        out_ref,
        myk_ref,
        myv_ref,
        mysc_ref,
        commk_ref,
        commv_ref,
        commsc_ref,
        send_sems,
        recv_sems,
    ):
        my = lax.axis_index("i")
        left = (my + N_DEV - 1) % N_DEV
        right = (my + 1) % N_DEV

        barrier_sem = pltpu.get_barrier_semaphore()
        for nbr in (left, right):
            pl.semaphore_signal(
                barrier_sem,
                inc=1,
                device_id=(nbr,),
                device_id_type=pl.DeviceIdType.MESH,
            )
        pl.semaphore_wait(barrier_sem, 2)

        def quantize(x):
            amax = jnp.maximum(
                jnp.max(jnp.abs(x), axis=0, keepdims=True), 1e-30
            )
            x8 = jnp.rint(x * (127.0 / amax)).astype(jnp.int8)
            return x8, amax * (1.0 / 127.0)

        k8, sck = quantize(k_ref[:, :])
        v8, scv = quantize(v_ref[:, :])
        myk_ref[:, :] = k8
        myv_ref[:, :] = v8
        mysc_ref[0, 0:1, :] = sck
        mysc_ref[1, 0:1, :] = scv

        A = pl.ds(0, half)
        B = pl.ds(half, half)

        def rdma(i, src, dst, dev):
            return pltpu.make_async_remote_copy(
                src_ref=src,
                dst_ref=dst,
                send_sem=send_sems.at[i],
                recv_sem=recv_sems.at[i],
                device_id=(dev,),
                device_id_type=pl.DeviceIdType.MESH,
            )

        t = [
            rdma(0, myk_ref.at[A, :], commk_ref.at[1, A, :], left),
            rdma(1, myv_ref.at[A, :], commv_ref.at[1, A, :], left),
            rdma(2, mysc_ref, commsc_ref.at[1], left),
            rdma(3, myk_ref.at[A, :], commk_ref.at[0, A, :], right),
            rdma(4, myv_ref.at[A, :], commv_ref.at[0, A, :], right),
            rdma(5, mysc_ref, commsc_ref.at[0], right),
            rdma(6, myk_ref.at[B, :], commk_ref.at[1, B, :], left),
            rdma(7, myv_ref.at[B, :], commv_ref.at[1, B, :], left),
            rdma(8, myk_ref.at[B, :], commk_ref.at[0, B, :], right),
            rdma(9, myv_ref.at[B, :], commv_ref.at[0, B, :], right),
        ]
        for ti in t:
            ti.start()

        scale = 1.0 / math.sqrt(d)
        q_blk = (q_ref[:, :] * scale).astype(jnp.bfloat16)

        def dequant(x8, sc):
            return (x8.astype(jnp.float32) * sc).astype(jnp.bfloat16)

        def attend(k_blk, v_blk, state):
            s = jnp.dot(q_blk, k_blk.T, preferred_element_type=jnp.float32)
            m_blk = jnp.max(s, axis=1, keepdims=True)
            if state is None:
                m = m_blk
                p = jnp.exp(s - m)
                l = jnp.sum(p, axis=1, keepdims=True)
                acc = jnp.dot(
                    p.astype(jnp.bfloat16),
                    v_blk,
                    preferred_element_type=jnp.float32,
                )
            else:
                m_prev, l_prev, acc_prev = state
                m = jnp.maximum(m_prev, m_blk)
                alpha = jnp.exp(m_prev - m)
                p = jnp.exp(s - m)
                l = l_prev * alpha + jnp.sum(p, axis=1, keepdims=True)
                acc = acc_prev * alpha + jnp.dot(
                    p.astype(jnp.bfloat16),
                    v_blk,
                    preferred_element_type=jnp.float32,
                )
            return m, l, acc

        state = attend(
            dequant(myk_ref[:, :], mysc_ref[0, 0:1, :]),
            dequant(myv_ref[:, :], mysc_ref[1, 0:1, :]),
            None,
        )

        t[0].wait()
        t[1].wait()
        t[2].wait()
        f = [
            rdma(10, commk_ref.at[1, A, :], commk_ref.at[2, A, :], left),
            rdma(11, commv_ref.at[1, A, :], commv_ref.at[2, A, :], left),
            rdma(12, commsc_ref.at[1], commsc_ref.at[2], left),
        ]
        for fi in f[:3]:
            fi.start()
        state = attend(
            dequant(commk_ref[1, A, :], commsc_ref[1, 0, 0:1, :]),
            dequant(commv_ref[1, A, :], commsc_ref[1, 1, 0:1, :]),
            state,
        )

        t[3].wait()
        t[4].wait()
        t[5].wait()
        state = attend(
            dequant(commk_ref[0, A, :], commsc_ref[0, 0, 0:1, :]),
            dequant(commv_ref[0, A, :], commsc_ref[0, 1, 0:1, :]),
            state,
        )

        t[8].wait()
        t[9].wait()
        f.append(rdma(13, commk_ref.at[0, B, :], commk_ref.at[2, B, :], right))
        f.append(rdma(14, commv_ref.at[0, B, :], commv_ref.at[2, B, :], right))
        for fi in f[3:]:
            fi.start()
        state = attend(
            dequant(commk_ref[0, B, :], commsc_ref[0, 0, 0:1, :]),
            dequant(commv_ref[0, B, :], commsc_ref[0, 1, 0:1, :]),
            state,
        )

        t[6].wait()
        t[7].wait()
        state = attend(
            dequant(commk_ref[1, B, :], commsc_ref[1, 0, 0:1, :]),
            dequant(commv_ref[1, B, :], commsc_ref[1, 1, 0:1, :]),
            state,
        )

        for fi in f:
            fi.wait()
        state = attend(
            dequant(commk_ref[2, :, :], commsc_ref[2, 0, 0:1, :]),
            dequant(commv_ref[2, :, :], commsc_ref[2, 1, 0:1, :]),
            state,
        )

        _, l, acc = state
        out_ref[:, :] = acc / l

    return pl.pallas_call(
        body,
        out_shape=jax.ShapeDtypeStruct((s_per, d), jnp.float32),
        in_specs=[
            pl.BlockSpec(memory_space=pltpu.VMEM),
            pl.BlockSpec(memory_space=pltpu.VMEM),
            pl.BlockSpec(memory_space=pltpu.VMEM),
        ],
        out_specs=pl.BlockSpec(memory_space=pltpu.VMEM),
        scratch_shapes=[
            pltpu.VMEM((s_per, d), jnp.int8),
            pltpu.VMEM((s_per, d), jnp.int8),
            pltpu.VMEM((2, sc_rows, d), jnp.float32),
            pltpu.VMEM((3, s_per, d), jnp.int8),
            pltpu.VMEM((3, s_per, d), jnp.int8),
            pltpu.VMEM((3, 2, sc_rows, d), jnp.float32),
            pltpu.SemaphoreType.DMA((15,)),
            pltpu.SemaphoreType.DMA((15,)),
        ],
        compiler_params=pltpu.CompilerParams(collective_id=0),
    )(q, k, v)


# device time: 17746 ns/iter; 1.0027x vs baseline; 1.0027x over previous
import math

import jax
import jax.numpy as jnp
from jax import lax
from jax.experimental import pallas as pl
from jax.experimental.pallas import tpu as pltpu

N_DEV = 4


def kernel(q, k, v):
    s_per, d = q.shape
    half = s_per // 2
    sc_rows = s_per // d

    def body(
        q_ref,
        k_ref,
        v_ref,
        out_ref,
        myk_ref,
        myv_ref,
        mysc_ref,
        commk_ref,
        commv_ref,
        commsc_ref,
        send_sems,
        recv_sems,
    ):
        my = lax.axis_index("i")
        left = (my + N_DEV - 1) % N_DEV
        right = (my + 1) % N_DEV

        barrier_sem = pltpu.get_barrier_semaphore()
        for nbr in (left, right):
            pl.semaphore_signal(
                barrier_sem,
                inc=1,
                device_id=(nbr,),
                device_id_type=pl.DeviceIdType.MESH,
            )
        pl.semaphore_wait(barrier_sem, 2)

        A = pl.ds(0, half)
        B = pl.ds(half, half)

        def amax_scale(x):
            amax = jnp.maximum(
                jnp.max(jnp.abs(x), axis=0, keepdims=True), 1e-30
            )
            return 127.0 / amax, amax * (1.0 / 127.0)

        k_f32 = k_ref[:, :]
        v_f32 = v_ref[:, :]
        invk, sck = amax_scale(k_f32)
        invv, scv = amax_scale(v_f32)
        mysc_ref[0, 0:1, :] = sck
        mysc_ref[1, 0:1, :] = scv
        myk_ref[A, :] = jnp.rint(k_f32[:half] * invk).astype(jnp.int8)
        myv_ref[A, :] = jnp.rint(v_f32[:half] * invv).astype(jnp.int8)

        def rdma(i, src, dst, dev):
            return pltpu.make_async_remote_copy(
                src_ref=src,
                dst_ref=dst,
                send_sem=send_sems.at[i],
                recv_sem=recv_sems.at[i],
                device_id=(dev,),
                device_id_type=pl.DeviceIdType.MESH,
            )

        t = [None] * 10
        t[0] = rdma(0, myk_ref.at[A, :], commk_ref.at[1, A, :], left)
        t[1] = rdma(1, myv_ref.at[A, :], commv_ref.at[1, A, :], left)
        t[2] = rdma(2, mysc_ref, commsc_ref.at[1], left)
        t[3] = rdma(3, myk_ref.at[A, :], commk_ref.at[0, A, :], right)
        t[4] = rdma(4, myv_ref.at[A, :], commv_ref.at[0, A, :], right)
        t[5] = rdma(5, mysc_ref, commsc_ref.at[0], right)
        for ti in t[:6]:
            ti.start()

        myk_ref[B, :] = jnp.rint(k_f32[half:] * invk).astype(jnp.int8)
        myv_ref[B, :] = jnp.rint(v_f32[half:] * invv).astype(jnp.int8)
        t[6] = rdma(6, myk_ref.at[B, :], commk_ref.at[1, B, :], left)
        t[7] = rdma(7, myv_ref.at[B, :], commv_ref.at[1, B, :], left)
        t[8] = rdma(8, myk_ref.at[B, :], commk_ref.at[0, B, :], right)
        t[9] = rdma(9, myv_ref.at[B, :], commv_ref.at[0, B, :], right)
        for ti in t[6:]:
            ti.start()

        scale = 1.0 / math.sqrt(d)
        q_blk = (q_ref[:, :] * scale).astype(jnp.bfloat16)

        def dequant(x8, sc):
            return x8.astype(jnp.bfloat16) * sc.astype(jnp.bfloat16)

        def attend(k_blk, v_blk, state):
            s = jnp.dot(q_blk, k_blk.T, preferred_element_type=jnp.float32)
            m_blk = jnp.max(s, axis=1, keepdims=True)
            if state is None:
                m = m_blk
                p = jnp.exp(s - m)
                l = jnp.sum(p, axis=1, keepdims=True)
                acc = jnp.dot(
                    p.astype(jnp.bfloat16),
                    v_blk,
                    preferred_element_type=jnp.float32,
                )
            else:
                m_prev, l_prev, acc_prev = state
                m = jnp.maximum(m_prev, m_blk)
                alpha = jnp.exp(m_prev - m)
                p = jnp.exp(s - m)
                l = l_prev * alpha + jnp.sum(p, axis=1, keepdims=True)
                acc = acc_prev * alpha + jnp.dot(
                    p.astype(jnp.bfloat16),
                    v_blk,
                    preferred_element_type=jnp.float32,
                )
            return m, l, acc

        state = attend(
            k_f32.astype(jnp.bfloat16), v_f32.astype(jnp.bfloat16), None
        )

        t[0].wait()
        t[1].wait()
        t[2].wait()
        f = [
            rdma(10, commk_ref.at[1, A, :], commk_ref.at[2, A, :], left),
            rdma(11, commv_ref.at[1, A, :], commv_ref.at[2, A, :], left),
            rdma(12, commsc_ref.at[1], commsc_ref.at[2], left),
        ]
        for fi in f[:3]:
            fi.start()
        state = attend(
            dequant(commk_ref[1, A, :], commsc_ref[1, 0, 0:1, :]),
            dequant(commv_ref[1, A, :], commsc_ref[1, 1, 0:1, :]),
            state,
        )

        t[3].wait()
        t[4].wait()
        t[5].wait()
        state = attend(
            dequant(commk_ref[0, A, :], commsc_ref[0, 0, 0:1, :]),
            dequant(commv_ref[0, A, :], commsc_ref[0, 1, 0:1, :]),
            state,
        )

        t[8].wait()
        t[9].wait()
        f.append(rdma(13, commk_ref.at[0, B, :], commk_ref.at[2, B, :], right))
        f.append(rdma(14, commv_ref.at[0, B, :], commv_ref.at[2, B, :], right))
        for fi in f[3:]:
            fi.start()
        state = attend(
            dequant(commk_ref[0, B, :], commsc_ref[0, 0, 0:1, :]),
            dequant(commv_ref[0, B, :], commsc_ref[0, 1, 0:1, :]),
            state,
        )

        t[6].wait()
        t[7].wait()
        state = attend(
            dequant(commk_ref[1, B, :], commsc_ref[1, 0, 0:1, :]),
            dequant(commv_ref[1, B, :], commsc_ref[1, 1, 0:1, :]),
            state,
        )

        for fi in f:
            fi.wait()
        state = attend(
            dequant(commk_ref[2, :, :], commsc_ref[2, 0, 0:1, :]),
            dequant(commv_ref[2, :, :], commsc_ref[2, 1, 0:1, :]),
            state,
        )

        _, l, acc = state
        out_ref[:, :] = acc / l

    return pl.pallas_call(
        body,
        out_shape=jax.ShapeDtypeStruct((s_per, d), jnp.float32),
        in_specs=[
            pl.BlockSpec(memory_space=pltpu.VMEM),
            pl.BlockSpec(memory_space=pltpu.VMEM),
            pl.BlockSpec(memory_space=pltpu.VMEM),
        ],
        out_specs=pl.BlockSpec(memory_space=pltpu.VMEM),
        scratch_shapes=[
            pltpu.VMEM((s_per, d), jnp.int8),
            pltpu.VMEM((s_per, d), jnp.int8),
            pltpu.VMEM((2, sc_rows, d), jnp.float32),
            pltpu.VMEM((3, s_per, d), jnp.int8),
            pltpu.VMEM((3, s_per, d), jnp.int8),
            pltpu.VMEM((3, 2, sc_rows, d), jnp.float32),
            pltpu.SemaphoreType.DMA((15,)),
            pltpu.SemaphoreType.DMA((15,)),
        ],
        compiler_params=pltpu.CompilerParams(collective_id=0),
    )(q, k, v)


# device time: 17638 ns/iter; 1.0088x vs baseline; 1.0061x over previous
import math

import jax
import jax.numpy as jnp
from jax import lax
from jax.experimental import pallas as pl
from jax.experimental.pallas import tpu as pltpu

N_DEV = 4


def kernel(q, k, v):
    s_per, d = q.shape
    half = s_per // 2
    sc_rows = s_per // d

    def body(
        q_ref,
        k_ref,
        v_ref,
        out_ref,
        myk_ref,
        myv_ref,
        mysc_ref,
        commk_ref,
        commv_ref,
        commsc_ref,
        send_sems,
        recv_sems,
    ):
        my = lax.axis_index("i")
        left = (my + N_DEV - 1) % N_DEV
        right = (my + 1) % N_DEV

        barrier_sem = pltpu.get_barrier_semaphore()
        for nbr in (left, right):
            pl.semaphore_signal(
                barrier_sem,
                inc=1,
                device_id=(nbr,),
                device_id_type=pl.DeviceIdType.MESH,
            )
        pl.semaphore_wait(barrier_sem, 2)

        A = pl.ds(0, half)
        B = pl.ds(half, half)

        def amax_scale(x):
            amax = jnp.maximum(
                jnp.max(jnp.abs(x), axis=0, keepdims=True), 1e-30
            )
            return 127.0 / amax, amax * (1.0 / 127.0)

        k_f32 = k_ref[:, :]
        v_f32 = v_ref[:, :]
        invk, sck = amax_scale(k_f32)
        invv, scv = amax_scale(v_f32)
        mysc_ref[0, 0:1, :] = sck
        mysc_ref[1, 0:1, :] = scv
        myk_ref[A, :] = jnp.rint(k_f32[:half] * invk).astype(jnp.int8)
        myv_ref[A, :] = jnp.rint(v_f32[:half] * invv).astype(jnp.int8)

        def rdma(i, src, dst, dev):
            return pltpu.make_async_remote_copy(
                src_ref=src,
                dst_ref=dst,
                send_sem=send_sems.at[i],
                recv_sem=recv_sems.at[i],
                device_id=(dev,),
                device_id_type=pl.DeviceIdType.MESH,
            )

        t = [None] * 10
        t[0] = rdma(0, myk_ref.at[A, :], commk_ref.at[1, A, :], left)
        t[1] = rdma(1, myv_ref.at[A, :], commv_ref.at[1, A, :], left)
        t[2] = rdma(2, mysc_ref, commsc_ref.at[1], left)
        t[3] = rdma(3, myk_ref.at[A, :], commk_ref.at[0, A, :], right)
        t[4] = rdma(4, myv_ref.at[A, :], commv_ref.at[0, A, :], right)
        t[5] = rdma(5, mysc_ref, commsc_ref.at[0], right)
        for ti in t[:6]:
            ti.start()

        myk_ref[B, :] = jnp.rint(k_f32[half:] * invk).astype(jnp.int8)
        myv_ref[B, :] = jnp.rint(v_f32[half:] * invv).astype(jnp.int8)
        t[6] = rdma(6, myk_ref.at[B, :], commk_ref.at[1, B, :], left)
        t[7] = rdma(7, myv_ref.at[B, :], commv_ref.at[1, B, :], left)
        t[8] = rdma(8, myk_ref.at[B, :], commk_ref.at[0, B, :], right)
        t[9] = rdma(9, myv_ref.at[B, :], commv_ref.at[0, B, :], right)
        for ti in t[6:]:
            ti.start()

        scale = 1.0 / math.sqrt(d)
        q_blk = (q_ref[:, :] * scale).astype(jnp.bfloat16)

        def dequant(x8, sc):
            return x8.astype(jnp.bfloat16) * sc.astype(jnp.bfloat16)

        def attend(k_blk, v_blk, state):
            s = jnp.dot(q_blk, k_blk.T, preferred_element_type=jnp.float32)
            p = jnp.exp(s)
            l_blk = jnp.sum(p, axis=1, keepdims=True)
            acc_blk = jnp.dot(
                p.astype(jnp.bfloat16), v_blk, preferred_element_type=jnp.float32
            )
            if state is None:
                return l_blk, acc_blk
            l_prev, acc_prev = state
            return l_prev + l_blk, acc_prev + acc_blk

        state = attend(
            k_f32.astype(jnp.bfloat16), v_f32.astype(jnp.bfloat16), None
        )

        t[0].wait()
        t[1].wait()
        t[2].wait()
        f = [
            rdma(10, commk_ref.at[1, A, :], commk_ref.at[2, A, :], left),
            rdma(11, commv_ref.at[1, A, :], commv_ref.at[2, A, :], left),
            rdma(12, commsc_ref.at[1], commsc_ref.at[2], left),
        ]
        for fi in f[:3]:
            fi.start()
        state = attend(
            dequant(commk_ref[1, A, :], commsc_ref[1, 0, 0:1, :]),
            dequant(commv_ref[1, A, :], commsc_ref[1, 1, 0:1, :]),
            state,
        )

        t[3].wait()
        t[4].wait()
        t[5].wait()
        state = attend(
            dequant(commk_ref[0, A, :], commsc_ref[0, 0, 0:1, :]),
            dequant(commv_ref[0, A, :], commsc_ref[0, 1, 0:1, :]),
            state,
        )

        t[8].wait()
        t[9].wait()
        f.append(rdma(13, commk_ref.at[0, B, :], commk_ref.at[2, B, :], right))
        f.append(rdma(14, commv_ref.at[0, B, :], commv_ref.at[2, B, :], right))
        for fi in f[3:]:
            fi.start()
        state = attend(
            dequant(commk_ref[0, B, :], commsc_ref[0, 0, 0:1, :]),
            dequant(commv_ref[0, B, :], commsc_ref[0, 1, 0:1, :]),
            state,
        )

        t[6].wait()
        t[7].wait()
        state = attend(
            dequant(commk_ref[1, B, :], commsc_ref[1, 0, 0:1, :]),
            dequant(commv_ref[1, B, :], commsc_ref[1, 1, 0:1, :]),
            state,
        )

        for fi in f:
            fi.wait()
        state = attend(
            dequant(commk_ref[2, :, :], commsc_ref[2, 0, 0:1, :]),
            dequant(commv_ref[2, :, :], commsc_ref[2, 1, 0:1, :]),
            state,
        )

        l, acc = state
        out_ref[:, :] = acc / l

    return pl.pallas_call(
        body,
        out_shape=jax.ShapeDtypeStruct((s_per, d), jnp.float32),
        in_specs=[
            pl.BlockSpec(memory_space=pltpu.VMEM),
            pl.BlockSpec(memory_space=pltpu.VMEM),
            pl.BlockSpec(memory_space=pltpu.VMEM),
        ],
        out_specs=pl.BlockSpec(memory_space=pltpu.VMEM),
        scratch_shapes=[
            pltpu.VMEM((s_per, d), jnp.int8),
            pltpu.VMEM((s_per, d), jnp.int8),
            pltpu.VMEM((2, sc_rows, d), jnp.float32),
            pltpu.VMEM((3, s_per, d), jnp.int8),
            pltpu.VMEM((3, s_per, d), jnp.int8),
            pltpu.VMEM((3, 2, sc_rows, d), jnp.float32),
            pltpu.SemaphoreType.DMA((15,)),
            pltpu.SemaphoreType.DMA((15,)),
        ],
        compiler_params=pltpu.CompilerParams(collective_id=0),
    )(q, k, v)


# device time: 16953 ns/iter; 1.0496x vs baseline; 1.0404x over previous
import math

import jax
import jax.numpy as jnp
from jax import lax
from jax.experimental import pallas as pl
from jax.experimental.pallas import tpu as pltpu

N_DEV = 4


def kernel(q, k, v):
    s_per, d = q.shape
    half = s_per // 2
    sc_rows = s_per // d

    def body(
        q_ref,
        k_ref,
        v_ref,
        out_ref,
        myk_ref,
        myv_ref,
        mysc_ref,
        commk_ref,
        commv_ref,
        commsc_ref,
        send_sems,
        recv_sems,
    ):
        my = lax.axis_index("i")
        left = (my + N_DEV - 1) % N_DEV
        right = (my + 1) % N_DEV

        barrier_sem = pltpu.get_barrier_semaphore()
        for nbr in (left, right):
            pl.semaphore_signal(
                barrier_sem,
                inc=1,
                device_id=(nbr,),
                device_id_type=pl.DeviceIdType.MESH,
            )

        A = pl.ds(0, half)
        B = pl.ds(half, half)

        def amax_scale(x):
            amax = jnp.maximum(
                jnp.max(jnp.abs(x), axis=0, keepdims=True), 1e-30
            )
            return 127.0 / amax, amax * (1.0 / 127.0)

        k_f32 = k_ref[:, :]
        v_f32 = v_ref[:, :]
        invk, sck = amax_scale(k_f32)
        invv, scv = amax_scale(v_f32)
        mysc_ref[0, 0:1, :] = sck
        mysc_ref[1, 0:1, :] = scv
        myk_ref[A, :] = jnp.rint(k_f32[:half] * invk).astype(jnp.int8)
        myv_ref[A, :] = jnp.rint(v_f32[:half] * invv).astype(jnp.int8)

        pl.semaphore_wait(barrier_sem, 2)

        def rdma(i, src, dst, dev):
            return pltpu.make_async_remote_copy(
                src_ref=src,
                dst_ref=dst,
                send_sem=send_sems.at[i],
                recv_sem=recv_sems.at[i],
                device_id=(dev,),
                device_id_type=pl.DeviceIdType.MESH,
            )

        t = [None] * 10
        t[0] = rdma(0, myk_ref.at[A, :], commk_ref.at[1, A, :], left)
        t[1] = rdma(1, myv_ref.at[A, :], commv_ref.at[1, A, :], left)
        t[2] = rdma(2, mysc_ref, commsc_ref.at[1], left)
        t[3] = rdma(3, myk_ref.at[A, :], commk_ref.at[0, A, :], right)
        t[4] = rdma(4, myv_ref.at[A, :], commv_ref.at[0, A, :], right)
        t[5] = rdma(5, mysc_ref, commsc_ref.at[0], right)
        for ti in t[:6]:
            ti.start()

        myk_ref[B, :] = jnp.rint(k_f32[half:] * invk).astype(jnp.int8)
        myv_ref[B, :] = jnp.rint(v_f32[half:] * invv).astype(jnp.int8)
        t[6] = rdma(6, myk_ref.at[B, :], commk_ref.at[1, B, :], left)
        t[7] = rdma(7, myv_ref.at[B, :], commv_ref.at[1, B, :], left)
        t[8] = rdma(8, myk_ref.at[B, :], commk_ref.at[0, B, :], right)
        t[9] = rdma(9, myv_ref.at[B, :], commv_ref.at[0, B, :], right)
        for ti in t[6:]:
            ti.start()

        scale = 1.0 / math.sqrt(d)
        q_blk = (q_ref[:, :] * scale).astype(jnp.bfloat16)

        def dequant(x8, sc):
            return x8.astype(jnp.bfloat16) * sc.astype(jnp.bfloat16)

        def attend(k_blk, v_blk, state):
            s = jnp.dot(q_blk, k_blk.T, preferred_element_type=jnp.float32)
            p = jnp.exp(s)
            l_blk = jnp.sum(p, axis=1, keepdims=True)
            acc_blk = jnp.dot(
                p.astype(jnp.bfloat16), v_blk, preferred_element_type=jnp.float32
            )
            if state is None:
                return l_blk, acc_blk
            l_prev, acc_prev = state
            return l_prev + l_blk, acc_prev + acc_blk

        state = attend(
            k_f32.astype(jnp.bfloat16), v_f32.astype(jnp.bfloat16), None
        )

        t[0].wait()
        t[1].wait()
        t[2].wait()
        f = [
            rdma(10, commk_ref.at[1, A, :], commk_ref.at[2, A, :], left),
            rdma(11, commv_ref.at[1, A, :], commv_ref.at[2, A, :], left),
            rdma(12, commsc_ref.at[1], commsc_ref.at[2], left),
        ]
        for fi in f[:3]:
            fi.start()
        state = attend(
            dequant(commk_ref[1, A, :], commsc_ref[1, 0, 0:1, :]),
            dequant(commv_ref[1, A, :], commsc_ref[1, 1, 0:1, :]),
            state,
        )

        t[3].wait()
        t[4].wait()
        t[5].wait()
        state = attend(
            dequant(commk_ref[0, A, :], commsc_ref[0, 0, 0:1, :]),
            dequant(commv_ref[0, A, :], commsc_ref[0, 1, 0:1, :]),
            state,
        )

        t[8].wait()
        t[9].wait()
        f.append(rdma(13, commk_ref.at[0, B, :], commk_ref.at[2, B, :], right))
        f.append(rdma(14, commv_ref.at[0, B, :], commv_ref.at[2, B, :], right))
        for fi in f[3:]:
            fi.start()
        state = attend(
            dequant(commk_ref[0, B, :], commsc_ref[0, 0, 0:1, :]),
            dequant(commv_ref[0, B, :], commsc_ref[0, 1, 0:1, :]),
            state,
        )

        t[6].wait()
        t[7].wait()
        state = attend(
            dequant(commk_ref[1, B, :], commsc_ref[1, 0, 0:1, :]),
            dequant(commv_ref[1, B, :], commsc_ref[1, 1, 0:1, :]),
            state,
        )

        f[0].wait()
        f[1].wait()
        f[2].wait()
        state = attend(
            dequant(commk_ref[2, A, :], commsc_ref[2, 0, 0:1, :]),
            dequant(commv_ref[2, A, :], commsc_ref[2, 1, 0:1, :]),
            state,
        )
        f[3].wait()
        f[4].wait()
        state = attend(
            dequant(commk_ref[2, B, :], commsc_ref[2, 0, 0:1, :]),
            dequant(commv_ref[2, B, :], commsc_ref[2, 1, 0:1, :]),
            state,
        )

        l, acc = state
        out_ref[:, :] = acc / l

    return pl.pallas_call(
        body,
        out_shape=jax.ShapeDtypeStruct((s_per, d), jnp.float32),
        in_specs=[
            pl.BlockSpec(memory_space=pltpu.VMEM),
            pl.BlockSpec(memory_space=pltpu.VMEM),
            pl.BlockSpec(memory_space=pltpu.VMEM),
        ],
        out_specs=pl.BlockSpec(memory_space=pltpu.VMEM),
        scratch_shapes=[
            pltpu.VMEM((s_per, d), jnp.int8),
            pltpu.VMEM((s_per, d), jnp.int8),
            pltpu.VMEM((2, sc_rows, d), jnp.float32),
            pltpu.VMEM((3, s_per, d), jnp.int8),
            pltpu.VMEM((3, s_per, d), jnp.int8),
            pltpu.VMEM((3, 2, sc_rows, d), jnp.float32),
            pltpu.SemaphoreType.DMA((15,)),
            pltpu.SemaphoreType.DMA((15,)),
        ],
        compiler_params=pltpu.CompilerParams(collective_id=0),
    )(q, k, v)


# device time: 16871 ns/iter; 1.0547x vs baseline; 1.0049x over previous
import math

import jax
import jax.numpy as jnp
from jax import lax
from jax.experimental import pallas as pl
from jax.experimental.pallas import tpu as pltpu

N_DEV = 4


def kernel(q, k, v):
    s_per, d = q.shape
    half = s_per // 2
    sc_rows = s_per // d

    def body(
        q_ref,
        k_ref,
        v_ref,
        out_ref,
        mykv_ref,
        mysc_ref,
        commkv_ref,
        commsc_ref,
        send_sems,
        recv_sems,
    ):
        my = lax.axis_index("i")
        left = (my + N_DEV - 1) % N_DEV
        right = (my + 1) % N_DEV

        barrier_sem = pltpu.get_barrier_semaphore()
        for nbr in (left, right):
            pl.semaphore_signal(
                barrier_sem,
                inc=1,
                device_id=(nbr,),
                device_id_type=pl.DeviceIdType.MESH,
            )

        A = pl.ds(0, half)
        B = pl.ds(half, half)

        def amax_scale(x):
            amax = jnp.maximum(
                jnp.max(jnp.abs(x), axis=0, keepdims=True), 1e-30
            )
            return 127.0 / amax, amax * (1.0 / 127.0)

        k_f32 = k_ref[:, :]
        v_f32 = v_ref[:, :]
        invk, sck = amax_scale(k_f32)
        invv, scv = amax_scale(v_f32)
        mysc_ref[0, 0:1, :] = sck
        mysc_ref[1, 0:1, :] = scv
        mykv_ref[0, A, :] = jnp.rint(k_f32[:half] * invk).astype(jnp.int8)
        mykv_ref[1, A, :] = jnp.rint(v_f32[:half] * invv).astype(jnp.int8)

        pl.semaphore_wait(barrier_sem, 2)

        def rdma(i, src, dst, dev):
            return pltpu.make_async_remote_copy(
                src_ref=src,
                dst_ref=dst,
                send_sem=send_sems.at[i],
                recv_sem=recv_sems.at[i],
                device_id=(dev,),
                device_id_type=pl.DeviceIdType.MESH,
            )

        t = [None] * 6
        t[0] = rdma(0, mykv_ref.at[:, A, :], commkv_ref.at[1, :, A, :], left)
        t[1] = rdma(1, mysc_ref, commsc_ref.at[1], left)
        t[2] = rdma(2, mykv_ref.at[:, A, :], commkv_ref.at[0, :, A, :], right)
        t[3] = rdma(3, mysc_ref, commsc_ref.at[0], right)
        for ti in t[:4]:
            ti.start()

        mykv_ref[0, B, :] = jnp.rint(k_f32[half:] * invk).astype(jnp.int8)
        mykv_ref[1, B, :] = jnp.rint(v_f32[half:] * invv).astype(jnp.int8)
        t[4] = rdma(4, mykv_ref.at[:, B, :], commkv_ref.at[1, :, B, :], left)
        t[5] = rdma(5, mykv_ref.at[:, B, :], commkv_ref.at[0, :, B, :], right)
        t[4].start()
        t[5].start()

        scale = 1.0 / math.sqrt(d)
        q_blk = (q_ref[:, :] * scale).astype(jnp.bfloat16)

        def dequant(x8, sc):
            return x8.astype(jnp.bfloat16) * sc.astype(jnp.bfloat16)

        def attend(k_blk, v_blk, state):
            s = jnp.dot(q_blk, k_blk.T, preferred_element_type=jnp.float32)
            p = jnp.exp(s)
            l_blk = jnp.sum(p, axis=1, keepdims=True)
            acc_blk = jnp.dot(
                p.astype(jnp.bfloat16), v_blk, preferred_element_type=jnp.float32
            )
            if state is None:
                return l_blk, acc_blk
            l_prev, acc_prev = state
            return l_prev + l_blk, acc_prev + acc_blk

        state = attend(
            k_f32.astype(jnp.bfloat16), v_f32.astype(jnp.bfloat16), None
        )

        t[0].wait()
        t[1].wait()
        f = [
            rdma(6, commkv_ref.at[1, :, A, :], commkv_ref.at[2, :, A, :], left),
            rdma(7, commsc_ref.at[1], commsc_ref.at[2], left),
        ]
        f[0].start()
        f[1].start()
        state = attend(
            dequant(commkv_ref[1, 0, A, :], commsc_ref[1, 0, 0:1, :]),
            dequant(commkv_ref[1, 1, A, :], commsc_ref[1, 1, 0:1, :]),
            state,
        )

        t[2].wait()
        t[3].wait()
        state = attend(
            dequant(commkv_ref[0, 0, A, :], commsc_ref[0, 0, 0:1, :]),
            dequant(commkv_ref[0, 1, A, :], commsc_ref[0, 1, 0:1, :]),
            state,
        )

        t[5].wait()
        f.append(
            rdma(8, commkv_ref.at[0, :, B, :], commkv_ref.at[2, :, B, :], right)
        )
        f[2].start()
        state = attend(
            dequant(commkv_ref[0, 0, B, :], commsc_ref[0, 0, 0:1, :]),
            dequant(commkv_ref[0, 1, B, :], commsc_ref[0, 1, 0:1, :]),
            state,
        )

        t[4].wait()
        state = attend(
            dequant(commkv_ref[1, 0, B, :], commsc_ref[1, 0, 0:1, :]),
            dequant(commkv_ref[1, 1, B, :], commsc_ref[1, 1, 0:1, :]),
            state,
        )

        f[0].wait()
        f[1].wait()
        state = attend(
            dequant(commkv_ref[2, 0, A, :], commsc_ref[2, 0, 0:1, :]),
            dequant(commkv_ref[2, 1, A, :], commsc_ref[2, 1, 0:1, :]),
            state,
        )
        f[2].wait()
        state = attend(
            dequant(commkv_ref[2, 0, B, :], commsc_ref[2, 0, 0:1, :]),
            dequant(commkv_ref[2, 1, B, :], commsc_ref[2, 1, 0:1, :]),
            state,
        )

        l, acc = state
        out_ref[:, :] = acc / l

    return pl.pallas_call(
        body,
        out_shape=jax.ShapeDtypeStruct((s_per, d), jnp.float32),
        in_specs=[
            pl.BlockSpec(memory_space=pltpu.VMEM),
            pl.BlockSpec(memory_space=pltpu.VMEM),
            pl.BlockSpec(memory_space=pltpu.VMEM),
        ],
        out_specs=pl.BlockSpec(memory_space=pltpu.VMEM),
        scratch_shapes=[
            pltpu.VMEM((2, s_per, d), jnp.int8),
            pltpu.VMEM((2, sc_rows, d), jnp.float32),
            pltpu.VMEM((3, 2, s_per, d), jnp.int8),
            pltpu.VMEM((3, 2, sc_rows, d), jnp.float32),
            pltpu.SemaphoreType.DMA((9,)),
            pltpu.SemaphoreType.DMA((9,)),
        ],
        compiler_params=pltpu.CompilerParams(collective_id=0),
    )(q, k, v)
